# Initial kernel scaffold; baseline (speedup 1.0000x reference)
import functools
import math

import jax
import jax.numpy as jnp
from jax import lax
from jax.experimental import pallas as pl
from jax.experimental.pallas import tpu as pltpu
from jax.experimental.pallas import tpu_sc as plsc

N = 10000
E = 320000
D = 128
NT = 4
ET = 8

NC = 2
NS = 16
NW = NC * NS
CE = E // NW
G = 80
NB = CE // G
LG = G // 16
NPAD = 10240
SL = NPAD // NS
RPT = N // NS

BN = 1000


def _qvke_body(x_ref, nt_ref, wq_ref, wk_ref, wv_ref, we_ref, musc_ref,
               q_ref, v_ref, ke_ref):
    xb = x_ref[...]
    nt = nt_ref[...]
    k = jnp.zeros_like(xb)
    q = jnp.zeros_like(xb)
    v = jnp.zeros_like(xb)
    for t in range(NT):
        m = nt == t
        q_t = jnp.dot(xb, wq_ref[t], preferred_element_type=jnp.float32)
        k_t = jnp.dot(xb, wk_ref[t], preferred_element_type=jnp.float32)
        v_t = jnp.dot(xb, wv_ref[t], preferred_element_type=jnp.float32)
        q = q + jnp.where(m, q_t, 0.0)
        v = v + jnp.where(m, v_t, 0.0)
        k = k + jnp.where(m, k_t, 0.0)
    q_ref[...] = q
    v_ref[...] = v
    for t in range(ET):
        ke_ref[t] = (
            jnp.dot(k, we_ref[t], preferred_element_type=jnp.float32)
            * musc_ref[t]
        )


def _qvke(x, nt2d, w_q, w_k, w_v, w_e, musc):
    return pl.pallas_call(
        _qvke_body,
        grid=(N // BN,),
        in_specs=[
            pl.BlockSpec((BN, D), lambda i: (i, 0)),
            pl.BlockSpec((BN, 1), lambda i: (i, 0)),
            pl.BlockSpec((NT, D, D), lambda i: (0, 0, 0)),
            pl.BlockSpec((NT, D, D), lambda i: (0, 0, 0)),
            pl.BlockSpec((NT, D, D), lambda i: (0, 0, 0)),
            pl.BlockSpec((ET, D, D), lambda i: (0, 0, 0)),
            pl.BlockSpec(memory_space=pltpu.SMEM),
        ],
        out_specs=[
            pl.BlockSpec((BN, D), lambda i: (i, 0)),
            pl.BlockSpec((BN, D), lambda i: (i, 0)),
            pl.BlockSpec((ET, BN, D), lambda i: (0, i, 0)),
        ],
        out_shape=[
            jax.ShapeDtypeStruct((N, D), jnp.float32),
            jax.ShapeDtypeStruct((N, D), jnp.float32),
            jax.ShapeDtypeStruct((ET, N, D), jnp.float32),
        ],
    )(x, nt2d, w_q, w_k, w_v, w_e, musc)


_SC_MESH = plsc.VectorSubcoreMesh(
    core_axis_name="c", subcore_axis_name="s", num_cores=NC, num_subcores=NS
)


@functools.partial(
    pl.kernel,
    out_type=[
        jax.ShapeDtypeStruct((E,), jnp.float32),
        jax.ShapeDtypeStruct((NW, NPAD), jnp.float32),
    ],
    mesh=_SC_MESH,
    scratch_types=[
        pltpu.VMEM((G,), jnp.int32),
        pltpu.VMEM((G,), jnp.int32),
        pltpu.VMEM((G,), jnp.int32),
        pltpu.VMEM((G,), jnp.int32),
        pltpu.VMEM((G,), jnp.float32),
        pltpu.VMEM((G, D), jnp.float32),
        pltpu.VMEM((G, D), jnp.float32),
        pltpu.VMEM((NPAD,), jnp.float32),
        pltpu.SemaphoreType.DMA,
    ],
)
def _scores_kernel(q_hbm, ke_hbm, src_hbm, dst_hbm, et_hbm,
                   ex_hbm, ssum_hbm,
                   dstb, srcb, etb, kidxb, exb, qrows, kerows, ssum_v, sem):
    c = lax.axis_index("c")
    s = lax.axis_index("s")
    wid = s * NC + c
    base = wid * CE

    def zero_body(i, _):
        ssum_v[pl.ds(i * 16, 16)] = jnp.zeros((16,), jnp.float32)
        return 0

    lax.fori_loop(0, NPAD // 16, zero_body, 0)

    def blk_body(b, _):
        off = base + b * G
        pltpu.sync_copy(dst_hbm.at[pl.ds(off, G)], dstb)
        pltpu.sync_copy(src_hbm.at[pl.ds(off, G)], srcb)
        pltpu.sync_copy(et_hbm.at[pl.ds(off, G)], etb)

        def kidx_body(g, _):
            sv = srcb[pl.ds(g * 16, 16)]
            ev = etb[pl.ds(g * 16, 16)]
            kidxb[pl.ds(g * 16, 16)] = ev * N + sv
            return 0

        lax.fori_loop(0, LG, kidx_body, 0)

        cp_q = pltpu.async_copy(q_hbm.at[dstb], qrows, sem)
        cp_k = pltpu.async_copy(ke_hbm.at[kidxb], kerows, sem)
        cp_q.wait()
        cp_k.wait()

        def grp_body(g, _):
            rows = jnp.full((16,), g * 16, jnp.int32) + lax.iota(jnp.int32, 16)

            def dot_body(d, acc):
                dc = jnp.full((16,), d, jnp.int32)
                vq = plsc.load_gather(qrows, [rows, dc])
                vk = plsc.load_gather(kerows, [rows, dc])
                return acc + vq * vk

            acc = lax.fori_loop(0, D, dot_body, jnp.zeros((16,), jnp.float32))
            ex = jnp.exp(acc)
            exb[pl.ds(g * 16, 16)] = ex
            dv = dstb[pl.ds(g * 16, 16)]
            plsc.addupdate_scatter(ssum_v, [dv], ex)
            return 0

        lax.fori_loop(0, LG, grp_body, 0)
        pltpu.sync_copy(exb, ex_hbm.at[pl.ds(off, G)])
        return 0

    lax.fori_loop(0, NB, blk_body, 0)
    pltpu.sync_copy(ssum_v, ssum_hbm.at[wid])


@functools.partial(
    pl.kernel,
    out_type=jax.ShapeDtypeStruct((NC, N, D), jnp.float32),
    mesh=_SC_MESH,
    scratch_types=[
        pltpu.VMEM((G,), jnp.int32),
        pltpu.VMEM((G,), jnp.int32),
        pltpu.VMEM((G,), jnp.float32),
        pltpu.VMEM((G,), jnp.float32),
        pltpu.VMEM((G, D), jnp.float32),
        pltpu.VMEM((NPAD,), jnp.float32),
        pltpu.VMEM((SL,), jnp.float32),
        pltpu.VMEM((SL,), jnp.float32),
        pltpu.VMEM_SHARED((NPAD,), jnp.float32),
        pltpu.VMEM_SHARED((N, D), jnp.float32),
        pltpu.SemaphoreType.DMA,
    ],
)
def _agg_kernel(v_hbm, src_hbm, dst_hbm, ex_hbm, ssump_hbm, outp_hbm,
                dstb, srcb, exb, wb, vrows, ssum_v, accsl, tmpsl,
                ssum_sh, out_sh, sem):
    c = lax.axis_index("c")
    s = lax.axis_index("s")
    wid = s * NC + c
    base = wid * CE
    soff = s * SL

    pltpu.sync_copy(ssump_hbm.at[0, pl.ds(soff, SL)], accsl)

    def comb_body(j, _):
        pltpu.sync_copy(ssump_hbm.at[j, pl.ds(soff, SL)], tmpsl)

        def add_body(i, _):
            ix = pl.ds(i * 16, 16)
            accsl[ix] = accsl[ix] + tmpsl[ix]
            return 0

        lax.fori_loop(0, SL // 16, add_body, 0)
        return 0

    lax.fori_loop(1, NW, comb_body, 0)
    pltpu.sync_copy(accsl, ssum_sh.at[pl.ds(soff, SL)])

    def vz_body(r, _):
        for j in range(D // 16):
            vrows[r, pl.ds(j * 16, 16)] = jnp.zeros((16,), jnp.float32)
        return 0

    lax.fori_loop(0, G, vz_body, 0)
    for i in range(RPT // G):
        pltpu.sync_copy(vrows, out_sh.at[pl.ds(s * RPT + i * G, G)])
    rem = RPT % G
    pltpu.sync_copy(
        vrows.at[pl.ds(0, rem)],
        out_sh.at[pl.ds(s * RPT + (RPT // G) * G, rem)],
    )
    plsc.subcore_barrier()
    pltpu.sync_copy(ssum_sh, ssum_v)

    def blk_body(b, _):
        off = base + b * G
        pltpu.sync_copy(dst_hbm.at[pl.ds(off, G)], dstb)
        pltpu.sync_copy(src_hbm.at[pl.ds(off, G)], srcb)
        pltpu.sync_copy(ex_hbm.at[pl.ds(off, G)], exb)
        pltpu.async_copy(v_hbm.at[srcb], vrows, sem).wait()

        def w_body(g, _):
            ix = pl.ds(g * 16, 16)
            ssv = plsc.load_gather(ssum_v, [dstb[ix]])
            wb[ix] = exb[ix] / (ssv + 1e-10)
            return 0

        lax.fori_loop(0, LG, w_body, 0)

        def scale_body(e, _):
            wv = plsc.load_gather(wb, [jnp.full((16,), e, jnp.int32)])
            for j in range(D // 16):
                ix = pl.ds(j * 16, 16)
                vrows[e, ix] = vrows[e, ix] * wv
            return 0

        lax.fori_loop(0, G, scale_body, 0)
        pltpu.sync_copy(vrows, out_sh.at[dstb], add=True)
        return 0

    lax.fori_loop(0, NB, blk_body, 0)
    plsc.subcore_barrier()

    rem = RPT % G
    for i in range(RPT // G):
        r0 = s * RPT + i * G
        pltpu.sync_copy(out_sh.at[pl.ds(r0, G)], vrows)
        pltpu.sync_copy(vrows, outp_hbm.at[c, pl.ds(r0, G)])
    r0 = s * RPT + (RPT // G) * G
    pltpu.sync_copy(out_sh.at[pl.ds(r0, rem)], vrows.at[pl.ds(0, rem)])
    pltpu.sync_copy(vrows.at[pl.ds(0, rem)], outp_hbm.at[c, pl.ds(r0, rem)])


def _sum2_body(p_ref, o_ref):
    o_ref[...] = p_ref[0] + p_ref[1]


def _sum2(outp):
    return pl.pallas_call(
        _sum2_body,
        grid=(N // BN,),
        in_specs=[pl.BlockSpec((NC, BN, D), lambda i: (0, i, 0))],
        out_specs=pl.BlockSpec((BN, D), lambda i: (i, 0)),
        out_shape=jax.ShapeDtypeStruct((N, D), jnp.float32),
    )(outp)


def kernel(x, edge_index, edge_type, node_type, W_Q, W_K, W_V, W_edge, mu):
    src = edge_index[0]
    dst = edge_index[1]
    nt2d = node_type.reshape(N, 1)
    musc = (mu / math.sqrt(D)).astype(jnp.float32)
    q, v, ke = _qvke(x, nt2d, W_Q, W_K, W_V, W_edge, musc)
    ke2 = ke.reshape(ET * N, D)
    ex, ssump = _scores_kernel(q, ke2, src, dst, edge_type)
    outp = _agg_kernel(v, src, dst, ex, ssump)
    return _sum2(outp)


# speedup vs baseline: 8.2335x; 8.2335x over previous
import functools
import math

import jax
import jax.numpy as jnp
from jax import lax
from jax.experimental import pallas as pl
from jax.experimental.pallas import tpu as pltpu
from jax.experimental.pallas import tpu_sc as plsc

N = 10000
E = 320000
D = 128
NT = 4
ET = 8

NC = 2
NS = 16
NW = NC * NS
CE = E // NW
G = 80
NB = CE // G
LG = G // 16
NPAD = 10240
SL = NPAD // NS
RH = 624

BN = 1000


def _qvke_body(x_ref, nt_ref, wq_ref, wk_ref, wv_ref, we_ref, musc_ref,
               q_ref, v_ref, ke_ref):
    xb = x_ref[...]
    nt = nt_ref[...]
    k = jnp.zeros_like(xb)
    q = jnp.zeros_like(xb)
    v = jnp.zeros_like(xb)
    for t in range(NT):
        m = nt == t
        q_t = jnp.dot(xb, wq_ref[t], preferred_element_type=jnp.float32)
        k_t = jnp.dot(xb, wk_ref[t], preferred_element_type=jnp.float32)
        v_t = jnp.dot(xb, wv_ref[t], preferred_element_type=jnp.float32)
        q = q + jnp.where(m, q_t, 0.0)
        v = v + jnp.where(m, v_t, 0.0)
        k = k + jnp.where(m, k_t, 0.0)
    q_ref[...] = q
    v_ref[...] = v
    for t in range(ET):
        ke_ref[t] = (
            jnp.dot(k, we_ref[t], preferred_element_type=jnp.float32)
            * musc_ref[t]
        )


def _qvke(x, nt2d, w_q, w_k, w_v, w_e, musc):
    return pl.pallas_call(
        _qvke_body,
        grid=(N // BN,),
        in_specs=[
            pl.BlockSpec((BN, D), lambda i: (i, 0)),
            pl.BlockSpec((BN, 1), lambda i: (i, 0)),
            pl.BlockSpec((NT, D, D), lambda i: (0, 0, 0)),
            pl.BlockSpec((NT, D, D), lambda i: (0, 0, 0)),
            pl.BlockSpec((NT, D, D), lambda i: (0, 0, 0)),
            pl.BlockSpec((ET, D, D), lambda i: (0, 0, 0)),
            pl.BlockSpec(memory_space=pltpu.SMEM),
        ],
        out_specs=[
            pl.BlockSpec((BN, D), lambda i: (i, 0)),
            pl.BlockSpec((BN, D), lambda i: (i, 0)),
            pl.BlockSpec((ET, BN, D), lambda i: (0, i, 0)),
        ],
        out_shape=[
            jax.ShapeDtypeStruct((N, D), jnp.float32),
            jax.ShapeDtypeStruct((N, D), jnp.float32),
            jax.ShapeDtypeStruct((ET, N, D), jnp.float32),
        ],
    )(x, nt2d, w_q, w_k, w_v, w_e, musc)


_SC_MESH = plsc.VectorSubcoreMesh(
    core_axis_name="c", subcore_axis_name="s", num_cores=NC, num_subcores=NS
)


@functools.partial(
    pl.kernel,
    out_type=[
        jax.ShapeDtypeStruct((E,), jnp.float32),
        jax.ShapeDtypeStruct((NW * NPAD,), jnp.float32),
    ],
    mesh=_SC_MESH,
    scratch_types=[
        pltpu.VMEM((G,), jnp.int32),
        pltpu.VMEM((G,), jnp.int32),
        pltpu.VMEM((G,), jnp.int32),
        pltpu.VMEM((G,), jnp.int32),
        pltpu.VMEM((G,), jnp.float32),
        pltpu.VMEM((G, D), jnp.float32),
        pltpu.VMEM((G, D), jnp.float32),
        pltpu.VMEM((NPAD,), jnp.float32),
        pltpu.SemaphoreType.DMA,
    ],
    compiler_params=pltpu.CompilerParams(needs_layout_passes=False),
)
def _scores_kernel(q_hbm, ke_hbm, src_hbm, dst_hbm, et_hbm,
                   ex_hbm, ssum_hbm,
                   dstb, srcb, etb, kidxb, exb, qrows, kerows, ssum_v, sem):
    c = lax.axis_index("c")
    s = lax.axis_index("s")
    wid = s * NC + c
    base = wid * CE

    def zero_body(i, _):
        ssum_v[pl.ds(i * 16, 16)] = jnp.zeros((16,), jnp.float32)
        return 0

    lax.fori_loop(0, NPAD // 16, zero_body, 0)

    def blk_body(b, _):
        off = base + b * G
        pltpu.sync_copy(dst_hbm.at[pl.ds(off, G)], dstb)
        pltpu.sync_copy(src_hbm.at[pl.ds(off, G)], srcb)
        pltpu.sync_copy(et_hbm.at[pl.ds(off, G)], etb)

        def kidx_body(g, _):
            sv = srcb[pl.ds(g * 16, 16)]
            ev = etb[pl.ds(g * 16, 16)]
            kidxb[pl.ds(g * 16, 16)] = ev * N + sv
            return 0

        lax.fori_loop(0, LG, kidx_body, 0)

        cp_q = pltpu.async_copy(q_hbm.at[dstb], qrows, sem)
        cp_k = pltpu.async_copy(ke_hbm.at[kidxb], kerows, sem)
        cp_q.wait()
        cp_k.wait()

        lanes = lax.iota(jnp.int32, 16)

        def grp_body(g, _):
            def edge_body(i, grp):
                e = g * 16 + i
                acc = qrows[e, pl.ds(0, 16)] * kerows[e, pl.ds(0, 16)]
                for j in range(1, D // 16):
                    ix = pl.ds(j * 16, 16)
                    acc = acc + qrows[e, ix] * kerows[e, ix]
                total = jnp.sum(acc)
                return jnp.where(lanes == i, total, grp)

            acc = lax.fori_loop(
                0, 16, edge_body, jnp.zeros((16,), jnp.float32)
            )
            ex = jnp.exp(acc)
            exb[pl.ds(g * 16, 16)] = ex
            dv = dstb[pl.ds(g * 16, 16)]
            plsc.addupdate_scatter(ssum_v, [dv], ex)
            return 0

        lax.fori_loop(0, LG, grp_body, 0)
        pltpu.sync_copy(exb, ex_hbm.at[pl.ds(off, G)])
        return 0

    lax.fori_loop(0, NB, blk_body, 0)
    pltpu.sync_copy(ssum_v, ssum_hbm.at[pl.ds(wid * NPAD, NPAD)])


@functools.partial(
    pl.kernel,
    out_type=jax.ShapeDtypeStruct((NC, N, D), jnp.float32),
    mesh=_SC_MESH,
    scratch_types=[
        pltpu.VMEM((G,), jnp.int32),
        pltpu.VMEM((G,), jnp.int32),
        pltpu.VMEM((G,), jnp.float32),
        pltpu.VMEM((G,), jnp.float32),
        pltpu.VMEM((G, D), jnp.float32),
        pltpu.VMEM((NPAD,), jnp.float32),
        pltpu.VMEM((SL,), jnp.float32),
        pltpu.VMEM((SL,), jnp.float32),
        pltpu.VMEM_SHARED((NPAD,), jnp.float32),
        pltpu.VMEM_SHARED((N, D), jnp.float32),
        pltpu.SemaphoreType.DMA,
    ],
    compiler_params=pltpu.CompilerParams(needs_layout_passes=False),
)
def _agg_kernel(v_hbm, src_hbm, dst_hbm, ex_hbm, ssump_hbm, outp_hbm,
                dstb, srcb, exb, wb, vrows, ssum_v, accsl, tmpsl,
                ssum_sh, out_sh, sem):
    c = lax.axis_index("c")
    s = lax.axis_index("s")
    wid = s * NC + c
    base = wid * CE
    soff = s * SL

    pltpu.sync_copy(ssump_hbm.at[pl.ds(soff, SL)], accsl)

    def comb_body(j, _):
        pltpu.sync_copy(ssump_hbm.at[pl.ds(j * NPAD + soff, SL)], tmpsl)

        def add_body(i, _):
            ix = pl.ds(i * 16, 16)
            accsl[ix] = accsl[ix] + tmpsl[ix]
            return 0

        lax.fori_loop(0, SL // 16, add_body, 0)
        return 0

    lax.fori_loop(1, NW, comb_body, 0)
    pltpu.sync_copy(accsl, ssum_sh.at[pl.ds(soff, SL)])

    def vz_body(r, _):
        for j in range(D // 16):
            vrows[r, pl.ds(j * 16, 16)] = jnp.zeros((16,), jnp.float32)
        return 0

    lax.fori_loop(0, G, vz_body, 0)
    row0 = s * RH
    for i in range(RH // G):
        pltpu.sync_copy(vrows, out_sh.at[pl.ds(row0 + i * G, G)])
    rem = RH % G
    pltpu.sync_copy(
        vrows.at[pl.ds(0, rem)],
        out_sh.at[pl.ds(row0 + (RH // G) * G, rem)],
    )

    @pl.when(s == NS - 1)
    def _zero_tail():
        pltpu.sync_copy(
            vrows.at[pl.ds(0, N - NS * RH)],
            out_sh.at[pl.ds(NS * RH, N - NS * RH)],
        )

    plsc.subcore_barrier()
    pltpu.sync_copy(ssum_sh, ssum_v)

    def blk_body(b, _):
        off = base + b * G
        pltpu.sync_copy(dst_hbm.at[pl.ds(off, G)], dstb)
        pltpu.sync_copy(src_hbm.at[pl.ds(off, G)], srcb)
        pltpu.sync_copy(ex_hbm.at[pl.ds(off, G)], exb)
        pltpu.async_copy(v_hbm.at[srcb], vrows, sem).wait()

        def w_body(g, _):
            ix = pl.ds(g * 16, 16)
            ssv = plsc.load_gather(ssum_v, [dstb[ix]])
            wb[ix] = exb[ix] / (ssv + 1e-10)
            return 0

        lax.fori_loop(0, LG, w_body, 0)

        def scale_body(e, _):
            wv = plsc.load_gather(wb, [jnp.full((16,), e, jnp.int32)])
            for j in range(D // 16):
                ix = pl.ds(j * 16, 16)
                vrows[e, ix] = vrows[e, ix] * wv
            return 0

        lax.fori_loop(0, G, scale_body, 0)
        pltpu.sync_copy(vrows, out_sh.at[dstb], add=True)
        return 0

    lax.fori_loop(0, NB, blk_body, 0)
    plsc.subcore_barrier()

    rem = RH % G
    for i in range(RH // G):
        r0 = s * RH + i * G
        pltpu.sync_copy(out_sh.at[pl.ds(r0, G)], vrows)
        pltpu.sync_copy(vrows, outp_hbm.at[c, pl.ds(r0, G)])
    r0 = s * RH + (RH // G) * G
    pltpu.sync_copy(out_sh.at[pl.ds(r0, rem)], vrows.at[pl.ds(0, rem)])
    pltpu.sync_copy(vrows.at[pl.ds(0, rem)], outp_hbm.at[c, pl.ds(r0, rem)])

    @pl.when(s == NS - 1)
    def _write_tail():
        nt_ = N - NS * RH
        pltpu.sync_copy(
            out_sh.at[pl.ds(NS * RH, nt_)], vrows.at[pl.ds(0, nt_)]
        )
        pltpu.sync_copy(
            vrows.at[pl.ds(0, nt_)], outp_hbm.at[c, pl.ds(NS * RH, nt_)]
        )


def _sum2_body(p_ref, o_ref):
    o_ref[...] = p_ref[0] + p_ref[1]


def _sum2(outp):
    return pl.pallas_call(
        _sum2_body,
        grid=(N // BN,),
        in_specs=[pl.BlockSpec((NC, BN, D), lambda i: (0, i, 0))],
        out_specs=pl.BlockSpec((BN, D), lambda i: (i, 0)),
        out_shape=jax.ShapeDtypeStruct((N, D), jnp.float32),
    )(outp)


def kernel(x, edge_index, edge_type, node_type, W_Q, W_K, W_V, W_edge, mu):
    src = edge_index[0]
    dst = edge_index[1]
    nt2d = node_type.reshape(N, 1)
    musc = (mu / math.sqrt(D)).astype(jnp.float32)
    q, v, ke = _qvke(x, nt2d, W_Q, W_K, W_V, W_edge, musc)
    ke2 = ke.reshape(ET * N, D)
    ex, ssump = _scores_kernel(q, ke2, src, dst, edge_type)
    outp = _agg_kernel(v, src, dst, ex, ssump)
    return _sum2(outp)
